# Initial kernel scaffold; baseline (speedup 1.0000x reference)
import jax
import jax.numpy as jnp
from jax.experimental import pallas as pl


def kernel(atom_fea, edge_idx, edge_fea, batch, distance, edge_vec, W, att, bias, bn_gamma, bn_beta, W1, b1, W2, b2):
    raise NotImplementedError("write your pallas kernel here")


# speedup vs baseline: 7.5751x; 7.5751x over previous
import functools

import jax
import jax.numpy as jnp
from jax import lax
from jax.experimental import pallas as pl
from jax.experimental.pallas import tpu as pltpu
from jax.experimental.pallas import tpu_sc as plsc

N, E, D, DE, H = 10000, 320000, 128, 16, 4
HD = H * D
NC, NS, LL = 2, 16, 16
NW = NC * NS
CHUNK = 80
PER_W = E // NW
NCH_W = PER_W // CHUNK
PER_T = E // NS
NCH_T = PER_T // CHUNK
ROWS_T = 1000
EB = 512
NEB = E // EB
NB = 2000
NNB = N // NB

_mesh = plsc.VectorSubcoreMesh(core_axis_name="c", subcore_axis_name="s",
                               num_cores=NC, num_subcores=NS)


def _sp(x):
    return jnp.maximum(x, 0.0) + jnp.log1p(jnp.exp(-jnp.abs(x)))


def _silu(x):
    return x * (1.0 / (1.0 + jnp.exp(-x)))


def _atom_mm_body(x_ref, w_ref, o_ref):
    o_ref[...] = jnp.dot(x_ref[...], w_ref[...],
                         preferred_element_type=jnp.float32)


def _tc_atom_mm(atom_fea, Wx):
    return pl.pallas_call(
        _atom_mm_body,
        grid=(NNB,),
        in_specs=[pl.BlockSpec((NB, D), lambda b: (b, 0)),
                  pl.BlockSpec((D, HD), lambda b: (0, 0))],
        out_specs=pl.BlockSpec((NB, HD), lambda b: (b, 0)),
        out_shape=jax.ShapeDtypeStruct((N, HD), jnp.float32),
    )(atom_fea, Wx)


def _alpha_body(gi_ref, gj_ref, ef_ref, we_ref, att1_ref, att2_ref,
                g2_ref, beta_ref, o_ref):
    et = jnp.dot(ef_ref[...], we_ref[...], preferred_element_type=jnp.float32)
    xi = _sp(gi_ref[...] + et)
    xj = _sp(gj_ref[...] + et)
    cols = []
    for h in range(H):
        a1 = att1_ref[h:h + 1, :]
        a2 = att2_ref[h:h + 1, :]
        d = (jnp.sum(xi[:, h * D:(h + 1) * D] * a1, axis=1, keepdims=True)
             + jnp.sum(xj[:, h * D:(h + 1) * D] * a2, axis=1, keepdims=True))
        cols.append(d)
    cols.append(jnp.zeros((EB, 16 - H), jnp.float32))
    draw = jnp.concatenate(cols, axis=1)
    alpha = _sp(_sp(draw) * g2_ref[...] + beta_ref[...])
    o_ref[...] = alpha


def _tc_alpha(G_i, G_j, edge_fea, We, att1, att2, g2, beta):
    return pl.pallas_call(
        _alpha_body,
        grid=(NEB,),
        in_specs=[pl.BlockSpec((EB, HD), lambda b: (b, 0)),
                  pl.BlockSpec((EB, HD), lambda b: (b, 0)),
                  pl.BlockSpec((EB, DE), lambda b: (b, 0)),
                  pl.BlockSpec((DE, HD), lambda b: (0, 0)),
                  pl.BlockSpec((H, D), lambda b: (0, 0)),
                  pl.BlockSpec((H, D), lambda b: (0, 0)),
                  pl.BlockSpec((1, 16), lambda b: (0, 0)),
                  pl.BlockSpec((1, 16), lambda b: (0, 0))],
        out_specs=pl.BlockSpec((EB, 16), lambda b: (b, 0)),
        out_shape=jax.ShapeDtypeStruct((E, 16), jnp.float32),
    )(G_i, G_j, edge_fea, We, att1, att2, g2, beta)


def _msum_body(gj_ref, ef_ref, af_ref, we_ref, o_ref):
    et = jnp.dot(ef_ref[...], we_ref[...], preferred_element_type=jnp.float32)
    xj = _sp(gj_ref[...] + et)
    af = af_ref[...]
    acc = xj[:, 0:D] * af[:, 0:1]
    for h in range(1, H):
        acc = acc + xj[:, h * D:(h + 1) * D] * af[:, h:h + 1]
    o_ref[...] = acc * (1.0 / H)


def _tc_msum(G_j, edge_fea, alphaf, We):
    return pl.pallas_call(
        _msum_body,
        grid=(NEB,),
        in_specs=[pl.BlockSpec((EB, HD), lambda b: (b, 0)),
                  pl.BlockSpec((EB, DE), lambda b: (b, 0)),
                  pl.BlockSpec((EB, 16), lambda b: (b, 0)),
                  pl.BlockSpec((DE, HD), lambda b: (0, 0))],
        out_specs=pl.BlockSpec((EB, D), lambda b: (b, 0)),
        out_shape=jax.ShapeDtypeStruct((E, D), jnp.float32),
    )(G_j, edge_fea, alphaf, We)


def _outuv_body(p_ref, bias_ref, w1a_ref, w1b_ref, b1_ref,
                out_ref, u_ref, v_ref):
    s = p_ref[0] + p_ref[1] + bias_ref[...]
    out_ref[...] = s
    u_ref[...] = jnp.dot(s, w1a_ref[...],
                         preferred_element_type=jnp.float32) + b1_ref[...]
    v_ref[...] = jnp.dot(s, w1b_ref[...], preferred_element_type=jnp.float32)


def _tc_outuv(P, bias2d, W1a, W1b, b1_2d):
    return pl.pallas_call(
        _outuv_body,
        grid=(NNB,),
        in_specs=[pl.BlockSpec((NC, NB, D), lambda b: (0, b, 0)),
                  pl.BlockSpec((1, D), lambda b: (0, 0)),
                  pl.BlockSpec((D, D), lambda b: (0, 0)),
                  pl.BlockSpec((D, D), lambda b: (0, 0)),
                  pl.BlockSpec((1, D), lambda b: (0, 0))],
        out_specs=[pl.BlockSpec((NB, D), lambda b: (b, 0)),
                   pl.BlockSpec((NB, D), lambda b: (b, 0)),
                   pl.BlockSpec((NB, D), lambda b: (b, 0))],
        out_shape=[jax.ShapeDtypeStruct((N, D), jnp.float32)] * 3,
    )(P, bias2d, W1a, W1b, b1_2d)


def _edge_body(ur_ref, vc_ref, ef_ref, w1c_ref, w2_ref, b2_ref, o_ref):
    pre = (ur_ref[...] + vc_ref[...]
           + jnp.dot(ef_ref[...], w1c_ref[...],
                     preferred_element_type=jnp.float32))
    hh = _silu(pre)
    t = jnp.dot(hh, w2_ref[...], preferred_element_type=jnp.float32) + b2_ref[...]
    o_ref[...] = _silu(t)


def _tc_edge(Ur, Vc, edge_fea, W1c, W2, b2_2d):
    return pl.pallas_call(
        _edge_body,
        grid=(NEB,),
        in_specs=[pl.BlockSpec((EB, D), lambda b: (b, 0)),
                  pl.BlockSpec((EB, D), lambda b: (b, 0)),
                  pl.BlockSpec((EB, DE), lambda b: (b, 0)),
                  pl.BlockSpec((DE, D), lambda b: (0, 0)),
                  pl.BlockSpec((D, DE), lambda b: (0, 0)),
                  pl.BlockSpec((1, DE), lambda b: (0, 0))],
        out_specs=pl.BlockSpec((EB, DE), lambda b: (b, 0)),
        out_shape=jax.ShapeDtypeStruct((E, DE), jnp.float32),
    )(Ur, Vc, edge_fea, W1c, W2, b2_2d)


def _make_gather2(width):
    @functools.partial(
        pl.kernel,
        out_type=[jax.ShapeDtypeStruct((E, width), jnp.float32)] * 2,
        mesh=_mesh,
        scratch_types=[
            pltpu.VMEM((CHUNK,), jnp.int32),
            pltpu.VMEM((CHUNK, width), jnp.float32),
            pltpu.SemaphoreType.DMA,
        ],
    )
    def k(tab_a, idx_a, tab_b, idx_b, out_a, out_b, idx_v, rows_v, sem):
        wid = lax.axis_index("s") * NC + lax.axis_index("c")
        base0 = wid * PER_W

        def body(ci, _):
            base = base0 + ci * CHUNK
            pltpu.sync_copy(idx_a.at[pl.ds(base, CHUNK)], idx_v)
            pltpu.async_copy(tab_a.at[idx_v], rows_v, sem).wait()
            pltpu.sync_copy(rows_v, out_a.at[pl.ds(base, CHUNK)])
            pltpu.sync_copy(idx_b.at[pl.ds(base, CHUNK)], idx_v)
            pltpu.async_copy(tab_b.at[idx_v], rows_v, sem).wait()
            pltpu.sync_copy(rows_v, out_b.at[pl.ds(base, CHUNK)])
            return _

        lax.fori_loop(0, NCH_W, body, 0)

    return k


@functools.partial(
    pl.kernel,
    out_type=jax.ShapeDtypeStruct((E, 16), jnp.float32),
    mesh=_mesh,
    scratch_types=[
        pltpu.VMEM((CHUNK,), jnp.int32),
        pltpu.VMEM((CHUNK, 16), jnp.float32),
        pltpu.VMEM((CHUNK, D), jnp.float32),
        pltpu.VMEM((CHUNK, D), jnp.float32),
        pltpu.VMEM_SHARED((N, D), jnp.float32),
    ],
)
def _sc_softmax_den(alpha16, idx_i, zeros128, out, idx_v, av, ev, dv, den_sp):
    c = lax.axis_index("c")
    t = lax.axis_index("s")

    @pl.when(t < N // ROWS_T)
    def _():
        pltpu.sync_copy(zeros128, den_sp.at[pl.ds(t * ROWS_T, ROWS_T)])

    pltpu.sync_copy(zeros128.at[pl.ds(0, CHUNK)], ev)
    plsc.subcore_barrier()

    def body_a(ci, _):
        base = t * PER_T + ci * CHUNK
        pltpu.sync_copy(idx_i.at[pl.ds(base, CHUNK)], idx_v)
        pltpu.sync_copy(alpha16.at[pl.ds(base, CHUNK)], av)

        def expo(r, __):
            ev[r, pl.ds(0, 16)] = jnp.exp(av[r])
            return __

        lax.fori_loop(0, CHUNK, expo, 0)
        pltpu.sync_copy(ev, den_sp.at[idx_v], add=True)
        return _

    lax.fori_loop(0, NCH_T, body_a, 0)
    plsc.subcore_barrier()

    wid = t * NC + c
    base0 = wid * PER_W

    def body_b(ci, _):
        base = base0 + ci * CHUNK
        pltpu.sync_copy(idx_i.at[pl.ds(base, CHUNK)], idx_v)
        pltpu.sync_copy(alpha16.at[pl.ds(base, CHUNK)], av)
        pltpu.sync_copy(den_sp.at[idx_v], dv)

        def norm(r, __):
            av[r] = jnp.exp(av[r]) / (dv[r, pl.ds(0, 16)] + 1e-16)
            return __

        lax.fori_loop(0, CHUNK, norm, 0)
        pltpu.sync_copy(av, out.at[pl.ds(base, CHUNK)])
        return _

    lax.fori_loop(0, NCH_W, body_b, 0)


@functools.partial(
    pl.kernel,
    out_type=jax.ShapeDtypeStruct((NC, N, D), jnp.float32),
    mesh=_mesh,
    scratch_types=[
        pltpu.VMEM((CHUNK,), jnp.int32),
        pltpu.VMEM((CHUNK, D), jnp.float32),
        pltpu.VMEM_SHARED((N, D), jnp.float32),
    ],
)
def _sc_aggr(msum, idx_i, zeros128, out, idx_v, rows_v, acc_sp):
    c = lax.axis_index("c")
    t = lax.axis_index("s")

    @pl.when(t < N // ROWS_T)
    def _():
        pltpu.sync_copy(zeros128, acc_sp.at[pl.ds(t * ROWS_T, ROWS_T)])

    plsc.subcore_barrier()

    base0 = c * (E // NC) + t * PER_W

    def body(ci, _):
        base = base0 + ci * CHUNK
        pltpu.sync_copy(idx_i.at[pl.ds(base, CHUNK)], idx_v)
        pltpu.sync_copy(msum.at[pl.ds(base, CHUNK)], rows_v)
        pltpu.sync_copy(rows_v, acc_sp.at[idx_v], add=True)
        return _

    lax.fori_loop(0, NCH_W, body, 0)
    plsc.subcore_barrier()

    @pl.when(t < N // ROWS_T)
    def _():
        pltpu.sync_copy(acc_sp.at[pl.ds(t * ROWS_T, ROWS_T)],
                        out.at[c, pl.ds(t * ROWS_T, ROWS_T)])


_gather2_512 = _make_gather2(HD)
_gather2_128 = _make_gather2(D)


def kernel(atom_fea, edge_idx, edge_fea, batch, distance, edge_vec,
           W, att, bias, bn_gamma, bn_beta, W1, b1, W2, b2):
    i = edge_idx[0]
    j = edge_idx[1]
    Wx = W[:D]
    We = W[D:]
    att1 = att[0, :, :D]
    att2 = att[0, :, D:]
    g2 = jnp.zeros((1, 16), jnp.float32).at[0, :H].set(
        bn_gamma / jnp.sqrt(1.0 + 1e-5))
    beta = jnp.zeros((1, 16), jnp.float32).at[0, :H].set(bn_beta)
    zeros128 = jnp.zeros((ROWS_T, D), jnp.float32)
    del batch, distance, edge_vec

    atom_t = _tc_atom_mm(atom_fea, Wx)
    G_i, G_j = _gather2_512(atom_t, i, atom_t, j)
    alpha16 = _tc_alpha(G_i, G_j, edge_fea, We, att1, att2, g2, beta)
    alphaf = _sc_softmax_den(alpha16, i, zeros128)
    msum = _tc_msum(G_j, edge_fea, alphaf, We)
    P = _sc_aggr(msum, i, zeros128)
    out, U, V = _tc_outuv(P, bias[None], W1[:D], W1[D:2 * D], b1[None])
    Ur, Vc = _gather2_128(U, i, V, j)
    e = _tc_edge(Ur, Vc, edge_fea, W1[2 * D:], W2, b2[None])
    return (out, e)


# speedup vs baseline: 8.6144x; 1.1372x over previous
import functools

import jax
import jax.numpy as jnp
from jax import lax
from jax.experimental import pallas as pl
from jax.experimental.pallas import tpu as pltpu
from jax.experimental.pallas import tpu_sc as plsc

N, E, D, DE, H = 10000, 320000, 128, 16, 4
HD = H * D
NC, NS, LL = 2, 16, 16
NW = NC * NS
CHUNK = 80
PER_W = E // NW
NCH_W = PER_W // CHUNK
PER_T = E // NS
NCH_T = PER_T // CHUNK
ROWS_T = 1000
EB = 512
NEB = E // EB
NB = 2000
NNB = N // NB

_mesh = plsc.VectorSubcoreMesh(core_axis_name="c", subcore_axis_name="s",
                               num_cores=NC, num_subcores=NS)


def _sp(x):
    return jnp.maximum(x, 0.0) + jnp.log1p(jnp.exp(-jnp.abs(x)))


def _silu(x):
    return x * (1.0 / (1.0 + jnp.exp(-x)))


def _alpha_body(ai_ref, aj_ref, ef_ref, wx_ref, we_ref, att1_ref, att2_ref,
                g2_ref, beta_ref, o_ref):
    et = jnp.dot(ef_ref[...], we_ref[...], preferred_element_type=jnp.float32)
    xi = _sp(jnp.dot(ai_ref[...], wx_ref[...],
                     preferred_element_type=jnp.float32) + et)
    xj = _sp(jnp.dot(aj_ref[...], wx_ref[...],
                     preferred_element_type=jnp.float32) + et)
    cols = []
    for h in range(H):
        a1 = att1_ref[h:h + 1, :]
        a2 = att2_ref[h:h + 1, :]
        d = (jnp.sum(xi[:, h * D:(h + 1) * D] * a1, axis=1, keepdims=True)
             + jnp.sum(xj[:, h * D:(h + 1) * D] * a2, axis=1, keepdims=True))
        cols.append(d)
    cols.append(jnp.zeros((EB, 16 - H), jnp.float32))
    draw = jnp.concatenate(cols, axis=1)
    alpha = _sp(_sp(draw) * g2_ref[...] + beta_ref[...])
    o_ref[...] = alpha


def _tc_alpha(A_i, A_j, edge_fea, Wx, We, att1, att2, g2, beta):
    return pl.pallas_call(
        _alpha_body,
        grid=(NEB,),
        in_specs=[pl.BlockSpec((EB, D), lambda b: (b, 0)),
                  pl.BlockSpec((EB, D), lambda b: (b, 0)),
                  pl.BlockSpec((EB, DE), lambda b: (b, 0)),
                  pl.BlockSpec((D, HD), lambda b: (0, 0)),
                  pl.BlockSpec((DE, HD), lambda b: (0, 0)),
                  pl.BlockSpec((H, D), lambda b: (0, 0)),
                  pl.BlockSpec((H, D), lambda b: (0, 0)),
                  pl.BlockSpec((1, 16), lambda b: (0, 0)),
                  pl.BlockSpec((1, 16), lambda b: (0, 0))],
        out_specs=pl.BlockSpec((EB, 16), lambda b: (b, 0)),
        out_shape=jax.ShapeDtypeStruct((E, 16), jnp.float32),
    )(A_i, A_j, edge_fea, Wx, We, att1, att2, g2, beta)


def _msum_body(aj_ref, ef_ref, af_ref, wx_ref, we_ref, o_ref):
    et = jnp.dot(ef_ref[...], we_ref[...], preferred_element_type=jnp.float32)
    xj = _sp(jnp.dot(aj_ref[...], wx_ref[...],
                     preferred_element_type=jnp.float32) + et)
    af = af_ref[...]
    acc = xj[:, 0:D] * af[:, 0:1]
    for h in range(1, H):
        acc = acc + xj[:, h * D:(h + 1) * D] * af[:, h:h + 1]
    o_ref[...] = acc * (1.0 / H)


def _tc_msum(A_j, edge_fea, alphaf, Wx, We):
    return pl.pallas_call(
        _msum_body,
        grid=(NEB,),
        in_specs=[pl.BlockSpec((EB, D), lambda b: (b, 0)),
                  pl.BlockSpec((EB, DE), lambda b: (b, 0)),
                  pl.BlockSpec((EB, 16), lambda b: (b, 0)),
                  pl.BlockSpec((D, HD), lambda b: (0, 0)),
                  pl.BlockSpec((DE, HD), lambda b: (0, 0))],
        out_specs=pl.BlockSpec((EB, D), lambda b: (b, 0)),
        out_shape=jax.ShapeDtypeStruct((E, D), jnp.float32),
    )(A_j, edge_fea, alphaf, Wx, We)


def _outuv_body(p_ref, bias_ref, w1a_ref, w1b_ref, b1_ref,
                out_ref, u_ref, v_ref):
    s = p_ref[0] + p_ref[1] + bias_ref[...]
    out_ref[...] = s
    u_ref[...] = jnp.dot(s, w1a_ref[...],
                         preferred_element_type=jnp.float32) + b1_ref[...]
    v_ref[...] = jnp.dot(s, w1b_ref[...], preferred_element_type=jnp.float32)


def _tc_outuv(P, bias2d, W1a, W1b, b1_2d):
    return pl.pallas_call(
        _outuv_body,
        grid=(NNB,),
        in_specs=[pl.BlockSpec((NC, NB, D), lambda b: (0, b, 0)),
                  pl.BlockSpec((1, D), lambda b: (0, 0)),
                  pl.BlockSpec((D, D), lambda b: (0, 0)),
                  pl.BlockSpec((D, D), lambda b: (0, 0)),
                  pl.BlockSpec((1, D), lambda b: (0, 0))],
        out_specs=[pl.BlockSpec((NB, D), lambda b: (b, 0)),
                   pl.BlockSpec((NB, D), lambda b: (b, 0)),
                   pl.BlockSpec((NB, D), lambda b: (b, 0))],
        out_shape=[jax.ShapeDtypeStruct((N, D), jnp.float32)] * 3,
    )(P, bias2d, W1a, W1b, b1_2d)


def _edge_body(ur_ref, vc_ref, ef_ref, w1c_ref, w2_ref, b2_ref, o_ref):
    pre = (ur_ref[...] + vc_ref[...]
           + jnp.dot(ef_ref[...], w1c_ref[...],
                     preferred_element_type=jnp.float32))
    hh = _silu(pre)
    t = jnp.dot(hh, w2_ref[...], preferred_element_type=jnp.float32) + b2_ref[...]
    o_ref[...] = _silu(t)


def _tc_edge(Ur, Vc, edge_fea, W1c, W2, b2_2d):
    return pl.pallas_call(
        _edge_body,
        grid=(NEB,),
        in_specs=[pl.BlockSpec((EB, D), lambda b: (b, 0)),
                  pl.BlockSpec((EB, D), lambda b: (b, 0)),
                  pl.BlockSpec((EB, DE), lambda b: (b, 0)),
                  pl.BlockSpec((DE, D), lambda b: (0, 0)),
                  pl.BlockSpec((D, DE), lambda b: (0, 0)),
                  pl.BlockSpec((1, DE), lambda b: (0, 0))],
        out_specs=pl.BlockSpec((EB, DE), lambda b: (b, 0)),
        out_shape=jax.ShapeDtypeStruct((E, DE), jnp.float32),
    )(Ur, Vc, edge_fea, W1c, W2, b2_2d)


def _make_gather2(width):
    @functools.partial(
        pl.kernel,
        out_type=[jax.ShapeDtypeStruct((E, width), jnp.float32)] * 2,
        mesh=_mesh,
        scratch_types=[
            pltpu.VMEM((CHUNK,), jnp.int32),
            pltpu.VMEM((CHUNK, width), jnp.float32),
            pltpu.SemaphoreType.DMA,
        ],
    )
    def k(tab_a, idx_a, tab_b, idx_b, out_a, out_b, idx_v, rows_v, sem):
        wid = lax.axis_index("s") * NC + lax.axis_index("c")
        base0 = wid * PER_W

        def body(ci, _):
            base = base0 + ci * CHUNK
            pltpu.sync_copy(idx_a.at[pl.ds(base, CHUNK)], idx_v)
            pltpu.async_copy(tab_a.at[idx_v], rows_v, sem).wait()
            pltpu.sync_copy(rows_v, out_a.at[pl.ds(base, CHUNK)])
            pltpu.sync_copy(idx_b.at[pl.ds(base, CHUNK)], idx_v)
            pltpu.async_copy(tab_b.at[idx_v], rows_v, sem).wait()
            pltpu.sync_copy(rows_v, out_b.at[pl.ds(base, CHUNK)])
            return _

        lax.fori_loop(0, NCH_W, body, 0)

    return k


@functools.partial(
    pl.kernel,
    out_type=jax.ShapeDtypeStruct((E, 16), jnp.float32),
    mesh=_mesh,
    scratch_types=[
        pltpu.VMEM((CHUNK,), jnp.int32),
        pltpu.VMEM((CHUNK, 16), jnp.float32),
        pltpu.VMEM((CHUNK, D), jnp.float32),
        pltpu.VMEM((CHUNK, D), jnp.float32),
        pltpu.VMEM_SHARED((N, D), jnp.float32),
    ],
)
def _sc_softmax_den(alpha16, idx_i, zeros128, out, idx_v, av, ev, dv, den_sp):
    c = lax.axis_index("c")
    t = lax.axis_index("s")

    @pl.when(t < N // ROWS_T)
    def _():
        pltpu.sync_copy(zeros128, den_sp.at[pl.ds(t * ROWS_T, ROWS_T)])

    pltpu.sync_copy(zeros128.at[pl.ds(0, CHUNK)], ev)
    plsc.subcore_barrier()

    def body_a(ci, _):
        base = t * PER_T + ci * CHUNK
        pltpu.sync_copy(idx_i.at[pl.ds(base, CHUNK)], idx_v)
        pltpu.sync_copy(alpha16.at[pl.ds(base, CHUNK)], av)

        def expo(r, __):
            ev[r, pl.ds(0, 16)] = jnp.exp(av[r])
            return __

        lax.fori_loop(0, CHUNK, expo, 0)
        pltpu.sync_copy(ev, den_sp.at[idx_v], add=True)
        return _

    lax.fori_loop(0, NCH_T, body_a, 0)
    plsc.subcore_barrier()

    wid = t * NC + c
    base0 = wid * PER_W

    def body_b(ci, _):
        base = base0 + ci * CHUNK
        pltpu.sync_copy(idx_i.at[pl.ds(base, CHUNK)], idx_v)
        pltpu.sync_copy(alpha16.at[pl.ds(base, CHUNK)], av)
        pltpu.sync_copy(den_sp.at[idx_v], dv)

        def norm(r, __):
            av[r] = jnp.exp(av[r]) / (dv[r, pl.ds(0, 16)] + 1e-16)
            return __

        lax.fori_loop(0, CHUNK, norm, 0)
        pltpu.sync_copy(av, out.at[pl.ds(base, CHUNK)])
        return _

    lax.fori_loop(0, NCH_W, body_b, 0)


@functools.partial(
    pl.kernel,
    out_type=jax.ShapeDtypeStruct((NC, N, D), jnp.float32),
    mesh=_mesh,
    scratch_types=[
        pltpu.VMEM((CHUNK,), jnp.int32),
        pltpu.VMEM((CHUNK, D), jnp.float32),
        pltpu.VMEM_SHARED((N, D), jnp.float32),
    ],
)
def _sc_aggr(msum, idx_i, zeros128, out, idx_v, rows_v, acc_sp):
    c = lax.axis_index("c")
    t = lax.axis_index("s")

    @pl.when(t < N // ROWS_T)
    def _():
        pltpu.sync_copy(zeros128, acc_sp.at[pl.ds(t * ROWS_T, ROWS_T)])

    plsc.subcore_barrier()

    base0 = c * (E // NC) + t * PER_W

    def body(ci, _):
        base = base0 + ci * CHUNK
        pltpu.sync_copy(idx_i.at[pl.ds(base, CHUNK)], idx_v)
        pltpu.sync_copy(msum.at[pl.ds(base, CHUNK)], rows_v)
        pltpu.sync_copy(rows_v, acc_sp.at[idx_v], add=True)
        return _

    lax.fori_loop(0, NCH_W, body, 0)
    plsc.subcore_barrier()

    @pl.when(t < N // ROWS_T)
    def _():
        pltpu.sync_copy(acc_sp.at[pl.ds(t * ROWS_T, ROWS_T)],
                        out.at[c, pl.ds(t * ROWS_T, ROWS_T)])


_gather2_128 = _make_gather2(D)


def kernel(atom_fea, edge_idx, edge_fea, batch, distance, edge_vec,
           W, att, bias, bn_gamma, bn_beta, W1, b1, W2, b2):
    i = edge_idx[0]
    j = edge_idx[1]
    Wx = W[:D]
    We = W[D:]
    att1 = att[0, :, :D]
    att2 = att[0, :, D:]
    g2 = jnp.zeros((1, 16), jnp.float32).at[0, :H].set(
        bn_gamma / jnp.sqrt(1.0 + 1e-5))
    beta = jnp.zeros((1, 16), jnp.float32).at[0, :H].set(bn_beta)
    zeros128 = jnp.zeros((ROWS_T, D), jnp.float32)
    del batch, distance, edge_vec

    A_i, A_j = _gather2_128(atom_fea, i, atom_fea, j)
    alpha16 = _tc_alpha(A_i, A_j, edge_fea, Wx, We, att1, att2, g2, beta)
    alphaf = _sc_softmax_den(alpha16, i, zeros128)
    msum = _tc_msum(A_j, edge_fea, alphaf, Wx, We)
    P = _sc_aggr(msum, i, zeros128)
    out, U, V = _tc_outuv(P, bias[None], W1[:D], W1[D:2 * D], b1[None])
    Ur, Vc = _gather2_128(U, i, V, j)
    e = _tc_edge(Ur, Vc, edge_fea, W1[2 * D:], W2, b2[None])
    return (out, e)
